# baseline (speedup 1.0000x reference)
import functools

import jax
import jax.numpy as jnp
from jax import lax
from jax.experimental import pallas as pl
from jax.experimental.pallas import tpu as pltpu
from jax.experimental.pallas import tpu_sc as plsc

B, C, K = 4096, 26, 1000
CP = 32
NC, NS = 2, 16
NW = NC * NS
RPW = B // NW
L = 16

_mesh = plsc.VectorSubcoreMesh(core_axis_name="c", subcore_axis_name="s")


@functools.partial(
    pl.kernel,
    mesh=_mesh,
    out_type=jax.ShapeDtypeStruct((B, C, K), jnp.int32),
    compiler_params=pltpu.CompilerParams(disable_bounds_checks=True),
    scratch_types=[
        pltpu.VMEM((RPW * CP,), jnp.int32),
        pltpu.VMEM((C, K), jnp.int32),
        pltpu.VMEM((C, K), jnp.int32),
        pltpu.SemaphoreType.DMA,
        pltpu.SemaphoreType.DMA,
    ],
)
def _onehot_sc(x_hbm, out_hbm, xl, buf_a, buf_b, sem_a, sem_b):
    wid = lax.axis_index("s") * NC + lax.axis_index("c")
    base = wid * RPW

    pltpu.sync_copy(x_hbm.at[pl.ds(base * CP, RPW * CP)], xl)

    zeros = jnp.zeros((L,), jnp.int32)
    iota = lax.iota(jnp.int32, L)

    def zfill_row(c, _):
        def zfill_chunk(j, _):
            o = pl.multiple_of(j * L, L)
            buf_a[c, pl.ds(o, L)] = zeros
            buf_b[c, pl.ds(o, L)] = zeros
            return 0
        return lax.fori_loop(0, (K + L - 1) // L, zfill_chunk, 0)

    lax.fori_loop(0, C, zfill_row, 0)

    def put(buf, r, make_window):
        o0 = pl.multiple_of(r * CP, L)
        v0 = xl[pl.ds(o0, L)]
        v1 = xl[pl.ds(o0 + L, L)]
        off0, lane0 = (v0 >> 4) << 4, v0 & (L - 1)
        off1, lane1 = (v1 >> 4) << 4, v1 & (L - 1)
        for c in range(C):
            off = (off0 if c < L else off1)[c % L]
            lane = (lane0 if c < L else lane1)[c % L]
            buf[c, pl.ds(pl.multiple_of(off, L), L)] = make_window(lane)

    def set_one(lane):
        return jnp.where(iota == lane, 1, 0).astype(jnp.int32)

    def set_zero(lane):
        return zeros

    def fire(buf, r, sem):
        put(buf, r, set_one)
        pltpu.async_copy(buf, out_hbm.at[base + r], sem)

    def drain_reset(buf, r_prev, sem):
        pltpu.make_async_copy(buf, out_hbm.at[base], sem).wait()
        put(buf, r_prev, set_zero)

    fire(buf_a, 0, sem_a)
    fire(buf_b, 1, sem_b)

    def step(g, _):
        drain_reset(buf_a, 2 * g - 2, sem_a)
        fire(buf_a, 2 * g, sem_a)
        drain_reset(buf_b, 2 * g - 1, sem_b)
        fire(buf_b, 2 * g + 1, sem_b)
        return 0

    lax.fori_loop(1, RPW // 2, step, 0)

    pltpu.make_async_copy(buf_a, out_hbm.at[base], sem_a).wait()
    pltpu.make_async_copy(buf_b, out_hbm.at[base], sem_b).wait()


def kernel(x):
    xp = jnp.pad(x, ((0, 0), (0, CP - C)))
    return _onehot_sc(xp.reshape(B * CP))


# speedup vs baseline: 4.2843x; 4.2843x over previous
import functools

import jax
import jax.numpy as jnp
from jax import lax
from jax.experimental import pallas as pl
from jax.experimental.pallas import tpu as pltpu
from jax.experimental.pallas import tpu_sc as plsc

B, C, K = 4096, 26, 1000
CP = 32
NC, NS = 2, 16
NW = NC * NS
BPW = B // NW
L = 16
KT = K // 8
G = 25
NG = KT // G
NU = C * NG

_mesh = plsc.VectorSubcoreMesh(core_axis_name="c", subcore_axis_name="s")


@functools.partial(
    pl.kernel,
    mesh=_mesh,
    out_type=jax.ShapeDtypeStruct((C, K, B), jnp.int32),
    compiler_params=pltpu.CompilerParams(
        needs_layout_passes=False, disable_bounds_checks=True),
    scratch_types=[
        pltpu.VMEM((BPW * CP,), jnp.int32),
        pltpu.VMEM((G, 8, 128), jnp.int32),
        pltpu.VMEM((G, 8, 128), jnp.int32),
        pltpu.SemaphoreType.DMA,
        pltpu.SemaphoreType.DMA,
    ],
)
def _onehot_sc(x_hbm, out_hbm, xl, buf_a, buf_b, sem_a, sem_b):
    wid = lax.axis_index("s") * NC + lax.axis_index("c")
    b0 = wid * BPW

    pltpu.sync_copy(x_hbm.at[pl.ds(b0 * CP, BPW * CP)], xl)

    zeros = jnp.zeros((L,), jnp.int32)
    ones = jnp.ones((L,), jnp.int32)
    iota = lax.iota(jnp.int32, L)

    def zfill(t, _):
        def zfill_sub(ci, _):
            def zfill_chunk(j, _):
                o = pl.multiple_of(j * L, L)
                buf_a[t, ci, pl.ds(o, L)] = zeros
                buf_b[t, ci, pl.ds(o, L)] = zeros
                return 0
            return lax.fori_loop(0, 128 // L, zfill_chunk, 0)
        return lax.fori_loop(0, 8, zfill_sub, 0)

    lax.fori_loop(0, G, zfill, 0)

    def scatter(buf, u, what):
        c = u // NG
        kt0 = (u % NG) * G

        def chunk(j, _):
            lanes = j * L + iota
            v = plsc.load_gather(xl, [lanes * CP + c])
            kt = v >> 3
            m = (kt >= kt0) & (kt < kt0 + G)
            plsc.store_scatter(buf, [kt - kt0, v & 7, lanes], what, mask=m)
            return 0

        lax.fori_loop(0, BPW // L, chunk, 0)

    def fire(buf, u, sem):
        scatter(buf, u, ones)
        c = u // NG
        kt0 = (u % NG) * G

        def issue(t, _):
            ks = pl.multiple_of((kt0 + t) * 8, 8)
            bs = pl.multiple_of(b0, 128)
            pltpu.async_copy(
                buf.at[t], out_hbm.at[c, pl.ds(ks, 8), pl.ds(bs, 128)], sem)
            return 0

        lax.fori_loop(0, G, issue, 0)

    def drain(buf, sem):
        def one(t, _):
            pltpu.make_async_copy(
                buf.at[0], out_hbm.at[0, pl.ds(0, 8), pl.ds(0, 128)],
                sem).wait()
            return 0
        lax.fori_loop(0, G, one, 0)

    fire(buf_a, 0, sem_a)
    fire(buf_b, 1, sem_b)

    def step(p, _):
        u = 2 * p
        drain(buf_a, sem_a)
        scatter(buf_a, u - 2, zeros)
        fire(buf_a, u, sem_a)
        drain(buf_b, sem_b)
        scatter(buf_b, u - 1, zeros)
        fire(buf_b, u + 1, sem_b)
        return 0

    lax.fori_loop(1, NU // 2, step, 0)

    drain(buf_a, sem_a)
    drain(buf_b, sem_b)


def kernel(x):
    xp = jnp.pad(x, ((0, 0), (0, CP - C)))
    out = _onehot_sc(xp.reshape(B * CP))
    return jnp.transpose(out, (2, 0, 1))


# speedup vs baseline: 4.3349x; 1.0118x over previous
import functools

import jax
import jax.numpy as jnp
from jax import lax
from jax.experimental import pallas as pl
from jax.experimental.pallas import tpu as pltpu
from jax.experimental.pallas import tpu_sc as plsc

B, C, K = 4096, 26, 1000
CP = 32
NC, NS = 2, 16
NW = NC * NS
BPW = B // NW
L = 16
KT = K // 8
G = 25
NG = KT // G
NU = C * NG

_mesh = plsc.VectorSubcoreMesh(core_axis_name="c", subcore_axis_name="s")


@functools.partial(
    pl.kernel,
    mesh=_mesh,
    out_type=jax.ShapeDtypeStruct((C, K, B), jnp.int32),
    compiler_params=pltpu.CompilerParams(
        needs_layout_passes=False, disable_bounds_checks=True),
    scratch_types=[
        pltpu.VMEM((BPW * CP,), jnp.int32),
        pltpu.VMEM((G * 8, 128), jnp.int32),
        pltpu.VMEM((G * 8, 128), jnp.int32),
        pltpu.SemaphoreType.DMA,
        pltpu.SemaphoreType.DMA,
    ],
)
def _onehot_sc(x_hbm, out_hbm, xl, buf_a, buf_b, sem_a, sem_b):
    wid = lax.axis_index("s") * NC + lax.axis_index("c")
    b0 = wid * BPW

    pltpu.sync_copy(x_hbm.at[pl.ds(b0 * CP, BPW * CP)], xl)

    zeros = jnp.zeros((L,), jnp.int32)
    ones = jnp.ones((L,), jnp.int32)
    iota = lax.iota(jnp.int32, L)

    def zfill(r, _):
        def zfill_chunk(j, _):
            o = pl.multiple_of(j * L, L)
            buf_a[r, pl.ds(o, L)] = zeros
            buf_b[r, pl.ds(o, L)] = zeros
            return 0
        return lax.fori_loop(0, 128 // L, zfill_chunk, 0)

    lax.fori_loop(0, G * 8, zfill, 0)

    def scatter(buf, u, what):
        c = u // NG
        kt0 = (u % NG) * G

        def chunk(j, _):
            lanes = j * L + iota
            v = plsc.load_gather(xl, [lanes * CP + c])
            kt = v >> 3
            m = (kt >= kt0) & (kt < kt0 + G)
            plsc.store_scatter(buf, [(kt - kt0) * 8 + (v & 7), lanes], what,
                               mask=m)
            return 0

        lax.fori_loop(0, BPW // L, chunk, 0)

    def fire(buf, u, sem):
        scatter(buf, u, ones)
        c = u // NG
        kt0 = (u % NG) * G

        def issue(t, _):
            ks = pl.multiple_of((kt0 + t) * 8, 8)
            bs = pl.multiple_of(b0, 128)
            pltpu.async_copy(
                buf.at[pl.ds(pl.multiple_of(t * 8, 8), 8)],
                out_hbm.at[c, pl.ds(ks, 8), pl.ds(bs, 128)], sem)
            return 0

        lax.fori_loop(0, G, issue, 0)

    def drain(buf, sem):
        pltpu.make_async_copy(
            out_hbm.at[0, pl.ds(0, G * 8), pl.ds(0, 128)], buf, sem).wait()

    fire(buf_a, 0, sem_a)
    fire(buf_b, 1, sem_b)

    def step(p, _):
        u = 2 * p
        drain(buf_a, sem_a)
        scatter(buf_a, u - 2, zeros)
        fire(buf_a, u, sem_a)
        drain(buf_b, sem_b)
        scatter(buf_b, u - 1, zeros)
        fire(buf_b, u + 1, sem_b)
        return 0

    lax.fori_loop(1, NU // 2, step, 0)

    drain(buf_a, sem_a)
    drain(buf_b, sem_b)


def kernel(x):
    xp = jnp.pad(x, ((0, 0), (0, CP - C)))
    out = _onehot_sc(xp.reshape(B * CP))
    return jnp.transpose(out, (2, 0, 1))
